# baseline (speedup 1.0000x reference)
import functools

import jax
import jax.numpy as jnp
from jax import lax
from jax.experimental import pallas as pl
from jax.experimental.pallas import tpu as pltpu
from jax.experimental.pallas import tpu_sc as plsc

N_NODES = 10000
N_EDGES = 320000
NC = 2
NS = 16
NW = NC * NS
B = 128
NB = 80
EPT = NB * B
ACC_ROWS = 10240
RPT = ACC_ROWS // NS

DEGW = 8
ROWS_BLK = 1000
GRID = N_NODES // ROWS_BLK


def _sc_mesh():
    return plsc.VectorSubcoreMesh(core_axis_name="c", subcore_axis_name="s")


@functools.partial(
    pl.kernel,
    out_type=jax.ShapeDtypeStruct((NC, ACC_ROWS, DEGW), jnp.float32),
    mesh=_sc_mesh(),
    compiler_params=pltpu.CompilerParams(use_tc_tiling_on_sc=False),
    scratch_types=[
        pltpu.VMEM((NB, B), jnp.int32),
        pltpu.VMEM((B, DEGW), jnp.float32),
        pltpu.VMEM_SHARED((ACC_ROWS, DEGW), jnp.float32),
    ],
)
def _deg_kernel(dst_hbm, ones_hbm, zeros_hbm, out_hbm, dst_v, ones_v, acc):
    cid = lax.axis_index("c")
    sid = lax.axis_index("s")
    wid = sid * NC + cid
    pltpu.sync_copy(zeros_hbm, acc.at[pl.ds(sid * RPT, RPT)])
    pltpu.sync_copy(dst_hbm.at[wid], dst_v)
    pltpu.sync_copy(ones_hbm, ones_v)
    plsc.subcore_barrier()

    def body(g, carry):
        pltpu.sync_copy(ones_v, acc.at[dst_v.at[g]], add=True)
        return carry

    lax.fori_loop(0, NB, body, 0)
    plsc.subcore_barrier()
    pltpu.sync_copy(acc.at[pl.ds(sid * RPT, RPT)],
                    out_hbm.at[cid, pl.ds(sid * RPT, RPT)])


def _make_agg_kernel(F, n_chunks):
    NBC = NB // n_chunks

    @functools.partial(
        pl.kernel,
        out_type=jax.ShapeDtypeStruct((NC, ACC_ROWS, F), jnp.float32),
        mesh=_sc_mesh(),
        compiler_params=pltpu.CompilerParams(use_tc_tiling_on_sc=False),
        scratch_types=[
            pltpu.VMEM((NBC, B), jnp.int32),
            pltpu.VMEM((NBC, B), jnp.int32),
            pltpu.VMEM((B, F), jnp.float32),
            pltpu.VMEM((B, F), jnp.float32),
            pltpu.VMEM_SHARED((ACC_ROWS, F), jnp.float32),
            pltpu.SemaphoreType.DMA,
            pltpu.SemaphoreType.DMA,
            pltpu.SemaphoreType.DMA,
            pltpu.SemaphoreType.DMA,
        ],
    )
    def agg_kernel(h_hbm, src_hbm, dst_hbm, zeros_hbm, out_hbm,
                   src_v, dst_v, rows0, rows1, acc,
                   gsem0, gsem1, ssem0, ssem1):
        cid = lax.axis_index("c")
        sid = lax.axis_index("s")
        wid = sid * NC + cid
        pltpu.sync_copy(zeros_hbm, acc.at[pl.ds(sid * RPT, RPT)])
        plsc.subcore_barrier()

        def start_gather(g, buf, sem):
            pltpu.async_copy(h_hbm.at[src_v.at[g]], buf, sem)

        def wait_gather(buf, sem):
            pltpu.make_async_copy(h_hbm.at[pl.ds(0, B)], buf, sem).wait()

        def start_scatter(g, buf, sem):
            return pltpu.async_copy(buf, acc.at[dst_v.at[g]], sem, add=True)

        for c in range(n_chunks):
            pltpu.sync_copy(src_hbm.at[wid, pl.ds(c * NBC, NBC)], src_v)
            pltpu.sync_copy(dst_hbm.at[wid, pl.ds(c * NBC, NBC)], dst_v)
            start_gather(0, rows0, gsem0)
            start_gather(1, rows1, gsem1)

            def body(i, carry):
                g = i * 2
                wait_gather(rows0, gsem0)
                d0 = start_scatter(g, rows0, ssem0)
                wait_gather(rows1, gsem1)
                d1 = start_scatter(g + 1, rows1, ssem1)
                d0.wait()
                start_gather(g + 2, rows0, gsem0)
                d1.wait()
                start_gather(g + 3, rows1, gsem1)
                return carry

            lax.fori_loop(0, NBC // 2 - 1, body, 0)
            wait_gather(rows0, gsem0)
            pltpu.sync_copy(rows0, acc.at[dst_v.at[NBC - 2]], add=True)
            wait_gather(rows1, gsem1)
            pltpu.sync_copy(rows1, acc.at[dst_v.at[NBC - 1]], add=True)

        plsc.subcore_barrier()
        pltpu.sync_copy(acc.at[pl.ds(sid * RPT, RPT)],
                        out_hbm.at[cid, pl.ds(sid * RPT, RPT)])

    return agg_kernel


_agg_128 = _make_agg_kernel(128, 2)
_agg_64 = _make_agg_kernel(64, 1)
_agg_40 = _make_agg_kernel(40, 1)


def _dinv_of(degp_ref):
    deg = degp_ref[0, :, 0:1] + degp_ref[1, :, 0:1] + 1.0
    return lax.rsqrt(deg)


def _l1_body(x_ref, w_ref, degp_ref, o_ref):
    dinv = _dinv_of(degp_ref)
    h = jnp.dot(x_ref[...], w_ref[...], preferred_element_type=jnp.float32)
    o_ref[...] = h * dinv


def _tc_layer1(x, W1, degp):
    return pl.pallas_call(
        _l1_body,
        grid=(GRID,),
        in_specs=[
            pl.BlockSpec((ROWS_BLK, 128), lambda i: (i, 0)),
            pl.BlockSpec((128, 128), lambda i: (0, 0)),
            pl.BlockSpec((NC, ROWS_BLK, DEGW), lambda i: (0, i, 0)),
        ],
        out_specs=pl.BlockSpec((ROWS_BLK, 128), lambda i: (i, 0)),
        out_shape=jax.ShapeDtypeStruct((N_NODES, 128), jnp.float32),
    )(x, W1, degp)


def _mid_body(part_ref, hp_ref, degp_ref, b_ref, w_ref, o_ref):
    dinv = _dinv_of(degp_ref)
    y = (part_ref[0] + part_ref[1] + hp_ref[...]) * dinv + b_ref[...]
    y = jnp.maximum(y, 0.0)
    o_ref[...] = jnp.dot(y, w_ref[...],
                         preferred_element_type=jnp.float32) * dinv


def _tc_mid(part, hp, degp, b, Wn):
    F = hp.shape[1]
    Fn = Wn.shape[1]
    return pl.pallas_call(
        _mid_body,
        grid=(GRID,),
        in_specs=[
            pl.BlockSpec((NC, ROWS_BLK, F), lambda i: (0, i, 0)),
            pl.BlockSpec((ROWS_BLK, F), lambda i: (i, 0)),
            pl.BlockSpec((NC, ROWS_BLK, DEGW), lambda i: (0, i, 0)),
            pl.BlockSpec((1, F), lambda i: (0, 0)),
            pl.BlockSpec((F, Fn), lambda i: (0, 0)),
        ],
        out_specs=pl.BlockSpec((ROWS_BLK, Fn), lambda i: (i, 0)),
        out_shape=jax.ShapeDtypeStruct((N_NODES, Fn), jnp.float32),
    )(part, hp, degp, b, Wn)


def _fin_body(part_ref, hp_ref, degp_ref, b_ref, o_ref):
    dinv = _dinv_of(degp_ref)
    y = (part_ref[0] + part_ref[1] + hp_ref[...]) * dinv + b_ref[...]
    m = jnp.max(y, axis=1, keepdims=True)
    z = y - m
    o_ref[...] = z - jnp.log(jnp.sum(jnp.exp(z), axis=1, keepdims=True))


def _tc_final(part, hp, degp, b):
    F = hp.shape[1]
    return pl.pallas_call(
        _fin_body,
        grid=(GRID,),
        in_specs=[
            pl.BlockSpec((NC, ROWS_BLK, F), lambda i: (0, i, 0)),
            pl.BlockSpec((ROWS_BLK, F), lambda i: (i, 0)),
            pl.BlockSpec((NC, ROWS_BLK, DEGW), lambda i: (0, i, 0)),
            pl.BlockSpec((1, F), lambda i: (0, 0)),
        ],
        out_specs=pl.BlockSpec((ROWS_BLK, F), lambda i: (i, 0)),
        out_shape=jax.ShapeDtypeStruct((N_NODES, F), jnp.float32),
    )(part, hp, degp, b)


def kernel(x, edge_index, W1, b1, W2, b2, W3, b3):
    src = edge_index[0].astype(jnp.int32)
    dst = edge_index[1].astype(jnp.int32)
    pad = NW * EPT - N_EDGES
    src_p = jnp.concatenate(
        [src, jnp.zeros((pad,), jnp.int32)]).reshape(NW, NB, B)
    dst_p = jnp.concatenate(
        [dst, jnp.full((pad,), ACC_ROWS - 1, jnp.int32)]).reshape(NW, NB, B)

    ones_b = jnp.ones((B, DEGW), jnp.float32)
    degp = _deg_kernel(dst_p, ones_b, jnp.zeros((RPT, DEGW), jnp.float32))

    h1p = _tc_layer1(x, W1, degp)
    p1 = _agg_128(h1p, src_p, dst_p, jnp.zeros((RPT, 128), jnp.float32))
    h2p = _tc_mid(p1, h1p, degp, b1.reshape(1, -1), W2)
    p2 = _agg_64(h2p, src_p, dst_p, jnp.zeros((RPT, 64), jnp.float32))
    h3p = _tc_mid(p2, h2p, degp, b2.reshape(1, -1), W3)
    p3 = _agg_40(h3p, src_p, dst_p, jnp.zeros((RPT, 40), jnp.float32))
    return _tc_final(p3, h3p, degp, b3.reshape(1, -1))


# speedup vs baseline: 1.6568x; 1.6568x over previous
import functools

import jax
import jax.numpy as jnp
from jax import lax
from jax.experimental import pallas as pl
from jax.experimental.pallas import tpu as pltpu
from jax.experimental.pallas import tpu_sc as plsc

N_NODES = 10000
N_EDGES = 320000
NC = 2
NS = 16
NW = NC * NS
B = 128
NB = 80
EPT = NB * B
ACC_ROWS = 10240
RPT = ACC_ROWS // NS

DEGW = 8
ROWS_BLK = 1000
GRID = N_NODES // ROWS_BLK


def _sc_mesh():
    return plsc.VectorSubcoreMesh(core_axis_name="c", subcore_axis_name="s")


@functools.partial(
    pl.kernel,
    out_type=jax.ShapeDtypeStruct((NC, ACC_ROWS, DEGW), jnp.float32),
    mesh=_sc_mesh(),
    compiler_params=pltpu.CompilerParams(use_tc_tiling_on_sc=False),
    scratch_types=[
        pltpu.VMEM((NB, B), jnp.int32),
        pltpu.VMEM((B, DEGW), jnp.float32),
        pltpu.VMEM_SHARED((ACC_ROWS, DEGW), jnp.float32),
    ],
)
def _deg_kernel(dst_hbm, ones_hbm, zeros_hbm, out_hbm, dst_v, ones_v, acc):
    cid = lax.axis_index("c")
    sid = lax.axis_index("s")
    wid = sid * NC + cid
    pltpu.sync_copy(zeros_hbm, acc.at[pl.ds(sid * RPT, RPT)])
    pltpu.sync_copy(dst_hbm.at[wid], dst_v)
    pltpu.sync_copy(ones_hbm, ones_v)
    plsc.subcore_barrier()

    def body(g, carry):
        pltpu.sync_copy(ones_v, acc.at[dst_v.at[g]], add=True)
        return carry

    lax.fori_loop(0, NB, body, 0)
    plsc.subcore_barrier()
    pltpu.sync_copy(acc.at[pl.ds(sid * RPT, RPT)],
                    out_hbm.at[cid, pl.ds(sid * RPT, RPT)])


def _make_agg_kernel(F, n_chunks):
    NBC = NB // n_chunks

    @functools.partial(
        pl.kernel,
        out_type=jax.ShapeDtypeStruct((NC, ACC_ROWS, F), jnp.bfloat16),
        mesh=_sc_mesh(),
        compiler_params=pltpu.CompilerParams(use_tc_tiling_on_sc=False),
        scratch_types=[
            pltpu.VMEM((NBC, B), jnp.int32),
            pltpu.VMEM((NBC, B), jnp.int32),
            pltpu.VMEM((B, F), jnp.bfloat16),
            pltpu.VMEM((B, F), jnp.bfloat16),
            pltpu.VMEM_SHARED((ACC_ROWS, F), jnp.bfloat16),
            pltpu.SemaphoreType.DMA,
            pltpu.SemaphoreType.DMA,
            pltpu.SemaphoreType.DMA,
            pltpu.SemaphoreType.DMA,
        ],
    )
    def agg_kernel(h_hbm, src_hbm, dst_hbm, zeros_hbm, out_hbm,
                   src_v, dst_v, rows0, rows1, acc,
                   gsem0, gsem1, ssem0, ssem1):
        cid = lax.axis_index("c")
        sid = lax.axis_index("s")
        wid = sid * NC + cid
        pltpu.sync_copy(zeros_hbm, acc.at[pl.ds(sid * RPT, RPT)])
        plsc.subcore_barrier()

        def start_gather(g, buf, sem):
            pltpu.async_copy(h_hbm.at[src_v.at[g]], buf, sem)

        def wait_gather(buf, sem):
            pltpu.make_async_copy(h_hbm.at[pl.ds(0, B)], buf, sem).wait()

        def start_scatter(g, buf, sem):
            return pltpu.async_copy(buf, acc.at[dst_v.at[g]], sem, add=True)

        for c in range(n_chunks):
            pltpu.sync_copy(src_hbm.at[wid, pl.ds(c * NBC, NBC)], src_v)
            pltpu.sync_copy(dst_hbm.at[wid, pl.ds(c * NBC, NBC)], dst_v)
            start_gather(0, rows0, gsem0)
            start_gather(1, rows1, gsem1)

            def body(i, carry):
                g = i * 2
                wait_gather(rows0, gsem0)
                d0 = start_scatter(g, rows0, ssem0)
                wait_gather(rows1, gsem1)
                d1 = start_scatter(g + 1, rows1, ssem1)
                d0.wait()
                start_gather(g + 2, rows0, gsem0)
                d1.wait()
                start_gather(g + 3, rows1, gsem1)
                return carry

            lax.fori_loop(0, NBC // 2 - 1, body, 0)
            wait_gather(rows0, gsem0)
            pltpu.sync_copy(rows0, acc.at[dst_v.at[NBC - 2]], add=True)
            wait_gather(rows1, gsem1)
            pltpu.sync_copy(rows1, acc.at[dst_v.at[NBC - 1]], add=True)

        plsc.subcore_barrier()
        pltpu.sync_copy(acc.at[pl.ds(sid * RPT, RPT)],
                        out_hbm.at[cid, pl.ds(sid * RPT, RPT)])

    return agg_kernel


_agg_128 = _make_agg_kernel(128, 1)
_agg_64 = _make_agg_kernel(64, 1)
_agg_48 = _make_agg_kernel(48, 1)


def _dinv_of(degp_ref):
    deg = degp_ref[0, :, 0:1] + degp_ref[1, :, 0:1] + 1.0
    return lax.rsqrt(deg)


def _l1_body(x_ref, w_ref, degp_ref, o_ref):
    dinv = _dinv_of(degp_ref)
    h = jnp.dot(x_ref[...], w_ref[...], preferred_element_type=jnp.float32)
    o_ref[...] = (h * dinv).astype(jnp.bfloat16)


def _tc_layer1(x, W1, degp):
    return pl.pallas_call(
        _l1_body,
        grid=(GRID,),
        in_specs=[
            pl.BlockSpec((ROWS_BLK, 128), lambda i: (i, 0)),
            pl.BlockSpec((128, 128), lambda i: (0, 0)),
            pl.BlockSpec((NC, ROWS_BLK, DEGW), lambda i: (0, i, 0)),
        ],
        out_specs=pl.BlockSpec((ROWS_BLK, 128), lambda i: (i, 0)),
        out_shape=jax.ShapeDtypeStruct((N_NODES, 128), jnp.bfloat16),
    )(x, W1, degp)


def _mid_body(part_ref, hp_ref, degp_ref, b_ref, w_ref, o_ref):
    dinv = _dinv_of(degp_ref)
    agg = (part_ref[0].astype(jnp.float32) + part_ref[1].astype(jnp.float32)
           + hp_ref[...].astype(jnp.float32))
    y = jnp.maximum(agg * dinv + b_ref[...], 0.0)
    o_ref[...] = (jnp.dot(y, w_ref[...], preferred_element_type=jnp.float32)
                  * dinv).astype(jnp.bfloat16)


def _tc_mid(part, hp, degp, b, Wn):
    F = hp.shape[1]
    Fn = Wn.shape[1]
    return pl.pallas_call(
        _mid_body,
        grid=(GRID,),
        in_specs=[
            pl.BlockSpec((NC, ROWS_BLK, F), lambda i: (0, i, 0)),
            pl.BlockSpec((ROWS_BLK, F), lambda i: (i, 0)),
            pl.BlockSpec((NC, ROWS_BLK, DEGW), lambda i: (0, i, 0)),
            pl.BlockSpec((1, F), lambda i: (0, 0)),
            pl.BlockSpec((F, Fn), lambda i: (0, 0)),
        ],
        out_specs=pl.BlockSpec((ROWS_BLK, Fn), lambda i: (i, 0)),
        out_shape=jax.ShapeDtypeStruct((N_NODES, Fn), jnp.bfloat16),
    )(part, hp, degp, b, Wn)


def _fin_body(part_ref, hp_ref, degp_ref, b_ref, o_ref):
    dinv = _dinv_of(degp_ref)
    agg = (part_ref[0].astype(jnp.float32) + part_ref[1].astype(jnp.float32)
           + hp_ref[...].astype(jnp.float32))
    y = (agg * dinv + b_ref[...])[:, :40]
    m = jnp.max(y, axis=1, keepdims=True)
    z = y - m
    o_ref[...] = z - jnp.log(jnp.sum(jnp.exp(z), axis=1, keepdims=True))


def _tc_final(part, hp, degp, b):
    F = hp.shape[1]
    return pl.pallas_call(
        _fin_body,
        grid=(GRID,),
        in_specs=[
            pl.BlockSpec((NC, ROWS_BLK, F), lambda i: (0, i, 0)),
            pl.BlockSpec((ROWS_BLK, F), lambda i: (i, 0)),
            pl.BlockSpec((NC, ROWS_BLK, DEGW), lambda i: (0, i, 0)),
            pl.BlockSpec((1, F), lambda i: (0, 0)),
        ],
        out_specs=pl.BlockSpec((ROWS_BLK, 40), lambda i: (i, 0)),
        out_shape=jax.ShapeDtypeStruct((N_NODES, 40), jnp.float32),
    )(part, hp, degp, b)


def kernel(x, edge_index, W1, b1, W2, b2, W3, b3):
    src = edge_index[0].astype(jnp.int32)
    dst = edge_index[1].astype(jnp.int32)
    pad = NW * EPT - N_EDGES
    src_p = jnp.concatenate(
        [src, jnp.zeros((pad,), jnp.int32)]).reshape(NW, NB, B)
    dst_p = jnp.concatenate(
        [dst, jnp.full((pad,), ACC_ROWS - 1, jnp.int32)]).reshape(NW, NB, B)

    ones_b = jnp.ones((B, DEGW), jnp.float32)
    degp = _deg_kernel(dst_p, ones_b, jnp.zeros((RPT, DEGW), jnp.float32))

    W3p = jnp.pad(W3, ((0, 0), (0, 8)))
    b3p = jnp.pad(b3, (0, 8))

    h1p = _tc_layer1(x, W1, degp)
    p1 = _agg_128(h1p, src_p, dst_p, jnp.zeros((RPT, 128), jnp.bfloat16))
    h2p = _tc_mid(p1, h1p, degp, b1.reshape(1, -1), W2)
    p2 = _agg_64(h2p, src_p, dst_p, jnp.zeros((RPT, 64), jnp.bfloat16))
    h3p = _tc_mid(p2, h2p, degp, b2.reshape(1, -1), W3p)
    p3 = _agg_48(h3p, src_p, dst_p, jnp.zeros((RPT, 48), jnp.bfloat16))
    return _tc_final(p3, h3p, degp, b3p.reshape(1, -1))


# speedup vs baseline: 1.7916x; 1.0813x over previous
import functools

import jax
import jax.numpy as jnp
from jax import lax
from jax.experimental import pallas as pl
from jax.experimental.pallas import tpu as pltpu
from jax.experimental.pallas import tpu_sc as plsc

N_NODES = 10000
N_EDGES = 320000
NC = 2
NS = 16
NW = NC * NS
B = 128
NB = 80
EPT = NB * B
ACC_ROWS = 10240
RPT = ACC_ROWS // NS

DEGW = 8
ROWS_BLK = 1000
GRID = N_NODES // ROWS_BLK


def _sc_mesh():
    return plsc.VectorSubcoreMesh(core_axis_name="c", subcore_axis_name="s")


@functools.partial(
    pl.kernel,
    out_type=jax.ShapeDtypeStruct((NC, ACC_ROWS, DEGW), jnp.float32),
    mesh=_sc_mesh(),
    compiler_params=pltpu.CompilerParams(use_tc_tiling_on_sc=False),
    scratch_types=[
        pltpu.VMEM((NB, B), jnp.int32),
        pltpu.VMEM((B, DEGW), jnp.float32),
        pltpu.VMEM_SHARED((ACC_ROWS, DEGW), jnp.float32),
    ],
)
def _deg_kernel(dst_hbm, ones_hbm, zeros_hbm, out_hbm, dst_v, ones_v, acc):
    cid = lax.axis_index("c")
    sid = lax.axis_index("s")
    wid = sid * NC + cid
    pltpu.sync_copy(zeros_hbm, acc.at[pl.ds(sid * RPT, RPT)])
    pltpu.sync_copy(dst_hbm.at[pl.ds(wid * NB, NB)], dst_v)
    pltpu.sync_copy(ones_hbm, ones_v)
    plsc.subcore_barrier()

    def body(g, carry):
        pltpu.sync_copy(ones_v, acc.at[dst_v.at[g]], add=True)
        return carry

    lax.fori_loop(0, NB, body, 0)
    plsc.subcore_barrier()
    pltpu.sync_copy(acc.at[pl.ds(sid * RPT, RPT)],
                    out_hbm.at[cid, pl.ds(sid * RPT, RPT)])


def _make_agg_kernel(F, NB0, NB1):
    @functools.partial(
        pl.kernel,
        out_type=jax.ShapeDtypeStruct((NC, ACC_ROWS, F), jnp.bfloat16),
        mesh=_sc_mesh(),
        compiler_params=pltpu.CompilerParams(use_tc_tiling_on_sc=False),
        scratch_types=[
            pltpu.VMEM((NB0, B), jnp.int32),
            pltpu.VMEM((NB0, B), jnp.int32),
            pltpu.VMEM((B, F), jnp.bfloat16),
            pltpu.VMEM((B, F), jnp.bfloat16),
            pltpu.VMEM_SHARED((ACC_ROWS, F), jnp.bfloat16),
            pltpu.SemaphoreType.DMA,
            pltpu.SemaphoreType.DMA,
            pltpu.SemaphoreType.DMA,
            pltpu.SemaphoreType.DMA,
        ],
    )
    def agg_kernel(h_hbm, src_hbm, dst_hbm, zeros_hbm, out_hbm,
                   src_v, dst_v, rows0, rows1, acc,
                   gsem0, gsem1, ssem0, ssem1):
        cid = lax.axis_index("c")
        sid = lax.axis_index("s")
        pltpu.sync_copy(zeros_hbm, acc.at[pl.ds(sid * RPT, RPT)])
        plsc.subcore_barrier()

        def start_gather(g, buf, sem):
            pltpu.async_copy(h_hbm.at[src_v.at[g]], buf, sem)

        def wait_gather(buf, sem):
            pltpu.make_async_copy(h_hbm.at[pl.ds(0, B)], buf, sem).wait()

        def start_scatter(g, buf, sem):
            return pltpu.async_copy(buf, acc.at[dst_v.at[g]], sem, add=True)

        def run(off, nb):
            pltpu.sync_copy(src_hbm.at[pl.ds(off, nb)],
                            src_v.at[pl.ds(0, nb)])
            pltpu.sync_copy(dst_hbm.at[pl.ds(off, nb)],
                            dst_v.at[pl.ds(0, nb)])
            start_gather(0, rows0, gsem0)
            start_gather(1, rows1, gsem1)

            def body(i, carry):
                g = i * 2
                wait_gather(rows0, gsem0)
                d0 = start_scatter(g, rows0, ssem0)
                wait_gather(rows1, gsem1)
                d1 = start_scatter(g + 1, rows1, ssem1)
                d0.wait()
                start_gather(g + 2, rows0, gsem0)
                d1.wait()
                start_gather(g + 3, rows1, gsem1)
                return carry

            lax.fori_loop(0, nb // 2 - 1, body, 0)
            wait_gather(rows0, gsem0)
            pltpu.sync_copy(rows0, acc.at[dst_v.at[nb - 2]], add=True)
            wait_gather(rows1, gsem1)
            pltpu.sync_copy(rows1, acc.at[dst_v.at[nb - 1]], add=True)

        @pl.when(cid == 0)
        def _():
            run(sid * NB0, NB0)

        @pl.when(cid != 0)
        def _():
            run(NS * NB0 + sid * NB1, NB1)

        plsc.subcore_barrier()
        pltpu.sync_copy(acc.at[pl.ds(sid * RPT, RPT)],
                        out_hbm.at[cid, pl.ds(sid * RPT, RPT)])

    return agg_kernel


_agg_128 = _make_agg_kernel(128, 116, 44)
_agg_64 = _make_agg_kernel(64, 104, 56)
_agg_48 = _make_agg_kernel(48, 102, 58)


def _dinv_of(degp_ref):
    deg = degp_ref[0, :, 0:1] + degp_ref[1, :, 0:1] + 1.0
    return lax.rsqrt(deg)


def _l1_body(x_ref, w_ref, degp_ref, o_ref):
    dinv = _dinv_of(degp_ref)
    h = jnp.dot(x_ref[...], w_ref[...], preferred_element_type=jnp.float32)
    o_ref[...] = (h * dinv).astype(jnp.bfloat16)


def _tc_layer1(x, W1, degp):
    return pl.pallas_call(
        _l1_body,
        grid=(GRID,),
        in_specs=[
            pl.BlockSpec((ROWS_BLK, 128), lambda i: (i, 0)),
            pl.BlockSpec((128, 128), lambda i: (0, 0)),
            pl.BlockSpec((NC, ROWS_BLK, DEGW), lambda i: (0, i, 0)),
        ],
        out_specs=pl.BlockSpec((ROWS_BLK, 128), lambda i: (i, 0)),
        out_shape=jax.ShapeDtypeStruct((N_NODES, 128), jnp.bfloat16),
    )(x, W1, degp)


def _mid_body(part_ref, hp_ref, degp_ref, b_ref, w_ref, o_ref):
    dinv = _dinv_of(degp_ref)
    agg = (part_ref[0].astype(jnp.float32) + part_ref[1].astype(jnp.float32)
           + hp_ref[...].astype(jnp.float32))
    y = jnp.maximum(agg * dinv + b_ref[...], 0.0)
    o_ref[...] = (jnp.dot(y, w_ref[...], preferred_element_type=jnp.float32)
                  * dinv).astype(jnp.bfloat16)


def _tc_mid(part, hp, degp, b, Wn):
    F = hp.shape[1]
    Fn = Wn.shape[1]
    return pl.pallas_call(
        _mid_body,
        grid=(GRID,),
        in_specs=[
            pl.BlockSpec((NC, ROWS_BLK, F), lambda i: (0, i, 0)),
            pl.BlockSpec((ROWS_BLK, F), lambda i: (i, 0)),
            pl.BlockSpec((NC, ROWS_BLK, DEGW), lambda i: (0, i, 0)),
            pl.BlockSpec((1, F), lambda i: (0, 0)),
            pl.BlockSpec((F, Fn), lambda i: (0, 0)),
        ],
        out_specs=pl.BlockSpec((ROWS_BLK, Fn), lambda i: (i, 0)),
        out_shape=jax.ShapeDtypeStruct((N_NODES, Fn), jnp.bfloat16),
    )(part, hp, degp, b, Wn)


def _fin_body(part_ref, hp_ref, degp_ref, b_ref, o_ref):
    dinv = _dinv_of(degp_ref)
    agg = (part_ref[0].astype(jnp.float32) + part_ref[1].astype(jnp.float32)
           + hp_ref[...].astype(jnp.float32))
    y = (agg * dinv + b_ref[...])[:, :40]
    m = jnp.max(y, axis=1, keepdims=True)
    z = y - m
    o_ref[...] = z - jnp.log(jnp.sum(jnp.exp(z), axis=1, keepdims=True))


def _tc_final(part, hp, degp, b):
    F = hp.shape[1]
    return pl.pallas_call(
        _fin_body,
        grid=(GRID,),
        in_specs=[
            pl.BlockSpec((NC, ROWS_BLK, F), lambda i: (0, i, 0)),
            pl.BlockSpec((ROWS_BLK, F), lambda i: (i, 0)),
            pl.BlockSpec((NC, ROWS_BLK, DEGW), lambda i: (0, i, 0)),
            pl.BlockSpec((1, F), lambda i: (0, 0)),
        ],
        out_specs=pl.BlockSpec((ROWS_BLK, 40), lambda i: (i, 0)),
        out_shape=jax.ShapeDtypeStruct((N_NODES, 40), jnp.float32),
    )(part, hp, degp, b)


def kernel(x, edge_index, W1, b1, W2, b2, W3, b3):
    src = edge_index[0].astype(jnp.int32)
    dst = edge_index[1].astype(jnp.int32)
    pad = NW * EPT - N_EDGES
    src_p = jnp.concatenate(
        [src, jnp.zeros((pad,), jnp.int32)]).reshape(NW * NB, B)
    dst_p = jnp.concatenate(
        [dst, jnp.full((pad,), ACC_ROWS - 1, jnp.int32)]).reshape(NW * NB, B)

    ones_b = jnp.ones((B, DEGW), jnp.float32)
    degp = _deg_kernel(dst_p, ones_b, jnp.zeros((RPT, DEGW), jnp.float32))

    W3p = jnp.pad(W3, ((0, 0), (0, 8)))
    b3p = jnp.pad(b3, (0, 8))

    h1p = _tc_layer1(x, W1, degp)
    p1 = _agg_128(h1p, src_p, dst_p, jnp.zeros((RPT, 128), jnp.bfloat16))
    h2p = _tc_mid(p1, h1p, degp, b1.reshape(1, -1), W2)
    p2 = _agg_64(h2p, src_p, dst_p, jnp.zeros((RPT, 64), jnp.bfloat16))
    h3p = _tc_mid(p2, h2p, degp, b2.reshape(1, -1), W3p)
    p3 = _agg_48(h3p, src_p, dst_p, jnp.zeros((RPT, 48), jnp.bfloat16))
    return _tc_final(p3, h3p, degp, b3p.reshape(1, -1))


# speedup vs baseline: 1.8235x; 1.0179x over previous
import functools

import jax
import jax.numpy as jnp
from jax import lax
from jax.experimental import pallas as pl
from jax.experimental.pallas import tpu as pltpu
from jax.experimental.pallas import tpu_sc as plsc

N_NODES = 10000
N_EDGES = 320000
NC = 2
NS = 16
NW = NC * NS
B = 128
NB = 80
EPT = NB * B
ACC_ROWS = 10240
RPT = ACC_ROWS // NS

DEGW = 8
ROWS_BLK = 1000
GRID = N_NODES // ROWS_BLK


def _sc_mesh():
    return plsc.VectorSubcoreMesh(core_axis_name="c", subcore_axis_name="s")


@functools.partial(
    pl.kernel,
    out_type=jax.ShapeDtypeStruct((NC, ACC_ROWS, DEGW), jnp.float32),
    mesh=_sc_mesh(),
    compiler_params=pltpu.CompilerParams(use_tc_tiling_on_sc=False),
    scratch_types=[
        pltpu.VMEM((NB, B), jnp.int32),
        pltpu.VMEM((B, DEGW), jnp.float32),
        pltpu.VMEM_SHARED((ACC_ROWS, DEGW), jnp.float32),
    ],
)
def _deg_kernel(dst_hbm, ones_hbm, zeros_hbm, out_hbm, dst_v, ones_v, acc):
    cid = lax.axis_index("c")
    sid = lax.axis_index("s")
    wid = sid * NC + cid
    pltpu.sync_copy(zeros_hbm, acc.at[pl.ds(sid * RPT, RPT)])
    pltpu.sync_copy(dst_hbm.at[pl.ds(wid * NB, NB)], dst_v)
    pltpu.sync_copy(ones_hbm, ones_v)
    plsc.subcore_barrier()

    def body(g, carry):
        pltpu.sync_copy(ones_v, acc.at[dst_v.at[g]], add=True)
        return carry

    lax.fori_loop(0, NB, body, 0)
    plsc.subcore_barrier()
    pltpu.sync_copy(acc.at[pl.ds(sid * RPT, RPT)],
                    out_hbm.at[cid, pl.ds(sid * RPT, RPT)])


def _make_agg_kernel(F, NB0, NB1):
    @functools.partial(
        pl.kernel,
        out_type=jax.ShapeDtypeStruct((NC, ACC_ROWS, F), jnp.bfloat16),
        mesh=_sc_mesh(),
        compiler_params=pltpu.CompilerParams(use_tc_tiling_on_sc=False),
        scratch_types=[
            pltpu.VMEM((NB0, B), jnp.int32),
            pltpu.VMEM((NB0, B), jnp.int32),
            pltpu.VMEM((B, F), jnp.bfloat16),
            pltpu.VMEM((B, F), jnp.bfloat16),
            pltpu.VMEM_SHARED((ACC_ROWS, F), jnp.bfloat16),
            pltpu.SemaphoreType.DMA,
            pltpu.SemaphoreType.DMA,
            pltpu.SemaphoreType.DMA,
            pltpu.SemaphoreType.DMA,
        ],
    )
    def agg_kernel(h_hbm, src_hbm, dst_hbm, zeros_hbm, out_hbm,
                   src_v, dst_v, rows0, rows1, acc,
                   gsem0, gsem1, ssem0, ssem1):
        cid = lax.axis_index("c")
        sid = lax.axis_index("s")
        pltpu.sync_copy(zeros_hbm, acc.at[pl.ds(sid * RPT, RPT)])
        plsc.subcore_barrier()

        def start_gather(g, buf, sem):
            pltpu.async_copy(h_hbm.at[src_v.at[g]], buf, sem)

        def wait_gather(buf, sem):
            pltpu.make_async_copy(h_hbm.at[pl.ds(0, B)], buf, sem).wait()

        def start_scatter(g, buf, sem):
            return pltpu.async_copy(buf, acc.at[dst_v.at[g]], sem, add=True)

        def run(off, nb):
            pltpu.sync_copy(src_hbm.at[pl.ds(off, nb)],
                            src_v.at[pl.ds(0, nb)])
            pltpu.sync_copy(dst_hbm.at[pl.ds(off, nb)],
                            dst_v.at[pl.ds(0, nb)])
            start_gather(0, rows0, gsem0)
            start_gather(1, rows1, gsem1)

            def body(i, carry):
                g = i * 2
                wait_gather(rows0, gsem0)
                d0 = start_scatter(g, rows0, ssem0)
                wait_gather(rows1, gsem1)
                d1 = start_scatter(g + 1, rows1, ssem1)
                d0.wait()
                start_gather(g + 2, rows0, gsem0)
                d1.wait()
                start_gather(g + 3, rows1, gsem1)
                return carry

            lax.fori_loop(0, nb // 2 - 1, body, 0)
            wait_gather(rows0, gsem0)
            pltpu.sync_copy(rows0, acc.at[dst_v.at[nb - 2]], add=True)
            wait_gather(rows1, gsem1)
            pltpu.sync_copy(rows1, acc.at[dst_v.at[nb - 1]], add=True)

        @pl.when(cid == 0)
        def _():
            run(sid * NB0, NB0)

        @pl.when(cid != 0)
        def _():
            run(NS * NB0 + sid * NB1, NB1)

        plsc.subcore_barrier()
        pltpu.sync_copy(acc.at[pl.ds(sid * RPT, RPT)],
                        out_hbm.at[cid, pl.ds(sid * RPT, RPT)])

    return agg_kernel


_agg_128 = _make_agg_kernel(128, 134, 26)
_agg_64 = _make_agg_kernel(64, 104, 56)
_agg_48 = _make_agg_kernel(48, 102, 58)


def _dinv_of(degp_ref):
    deg = degp_ref[0, :, 0:1] + degp_ref[1, :, 0:1] + 1.0
    return lax.rsqrt(deg)


def _l1_body(x_ref, w_ref, degp_ref, o_ref):
    dinv = _dinv_of(degp_ref)
    h = jnp.dot(x_ref[...], w_ref[...], preferred_element_type=jnp.float32)
    o_ref[...] = (h * dinv).astype(jnp.bfloat16)


def _tc_layer1(x, W1, degp):
    return pl.pallas_call(
        _l1_body,
        grid=(GRID,),
        in_specs=[
            pl.BlockSpec((ROWS_BLK, 128), lambda i: (i, 0)),
            pl.BlockSpec((128, 128), lambda i: (0, 0)),
            pl.BlockSpec((NC, ROWS_BLK, DEGW), lambda i: (0, i, 0)),
        ],
        out_specs=pl.BlockSpec((ROWS_BLK, 128), lambda i: (i, 0)),
        out_shape=jax.ShapeDtypeStruct((N_NODES, 128), jnp.bfloat16),
    )(x, W1, degp)


def _mid_body(part_ref, hp_ref, degp_ref, b_ref, w_ref, o_ref):
    dinv = _dinv_of(degp_ref)
    agg = (part_ref[0].astype(jnp.float32) + part_ref[1].astype(jnp.float32)
           + hp_ref[...].astype(jnp.float32))
    y = jnp.maximum(agg * dinv + b_ref[...], 0.0)
    o_ref[...] = (jnp.dot(y, w_ref[...], preferred_element_type=jnp.float32)
                  * dinv).astype(jnp.bfloat16)


def _tc_mid(part, hp, degp, b, Wn):
    F = hp.shape[1]
    Fn = Wn.shape[1]
    return pl.pallas_call(
        _mid_body,
        grid=(GRID,),
        in_specs=[
            pl.BlockSpec((NC, ROWS_BLK, F), lambda i: (0, i, 0)),
            pl.BlockSpec((ROWS_BLK, F), lambda i: (i, 0)),
            pl.BlockSpec((NC, ROWS_BLK, DEGW), lambda i: (0, i, 0)),
            pl.BlockSpec((1, F), lambda i: (0, 0)),
            pl.BlockSpec((F, Fn), lambda i: (0, 0)),
        ],
        out_specs=pl.BlockSpec((ROWS_BLK, Fn), lambda i: (i, 0)),
        out_shape=jax.ShapeDtypeStruct((N_NODES, Fn), jnp.bfloat16),
    )(part, hp, degp, b, Wn)


def _fin_body(part_ref, hp_ref, degp_ref, b_ref, o_ref):
    dinv = _dinv_of(degp_ref)
    agg = (part_ref[0].astype(jnp.float32) + part_ref[1].astype(jnp.float32)
           + hp_ref[...].astype(jnp.float32))
    y = (agg * dinv + b_ref[...])[:, :40]
    m = jnp.max(y, axis=1, keepdims=True)
    z = y - m
    o_ref[...] = z - jnp.log(jnp.sum(jnp.exp(z), axis=1, keepdims=True))


def _tc_final(part, hp, degp, b):
    F = hp.shape[1]
    return pl.pallas_call(
        _fin_body,
        grid=(GRID,),
        in_specs=[
            pl.BlockSpec((NC, ROWS_BLK, F), lambda i: (0, i, 0)),
            pl.BlockSpec((ROWS_BLK, F), lambda i: (i, 0)),
            pl.BlockSpec((NC, ROWS_BLK, DEGW), lambda i: (0, i, 0)),
            pl.BlockSpec((1, F), lambda i: (0, 0)),
        ],
        out_specs=pl.BlockSpec((ROWS_BLK, 40), lambda i: (i, 0)),
        out_shape=jax.ShapeDtypeStruct((N_NODES, 40), jnp.float32),
    )(part, hp, degp, b)


def kernel(x, edge_index, W1, b1, W2, b2, W3, b3):
    src = edge_index[0].astype(jnp.int32)
    dst = edge_index[1].astype(jnp.int32)
    pad = NW * EPT - N_EDGES
    src_p = jnp.concatenate(
        [src, jnp.zeros((pad,), jnp.int32)]).reshape(NW * NB, B)
    dst_p = jnp.concatenate(
        [dst, jnp.full((pad,), ACC_ROWS - 1, jnp.int32)]).reshape(NW * NB, B)

    ones_b = jnp.ones((B, DEGW), jnp.float32)
    degp = _deg_kernel(dst_p, ones_b, jnp.zeros((RPT, DEGW), jnp.float32))

    W3p = jnp.pad(W3, ((0, 0), (0, 8)))
    b3p = jnp.pad(b3, (0, 8))

    h1p = _tc_layer1(x, W1, degp)
    p1 = _agg_128(h1p, src_p, dst_p, jnp.zeros((RPT, 128), jnp.bfloat16))
    h2p = _tc_mid(p1, h1p, degp, b1.reshape(1, -1), W2)
    p2 = _agg_64(h2p, src_p, dst_p, jnp.zeros((RPT, 64), jnp.bfloat16))
    h3p = _tc_mid(p2, h2p, degp, b2.reshape(1, -1), W3p)
    p3 = _agg_48(h3p, src_p, dst_p, jnp.zeros((RPT, 48), jnp.bfloat16))
    return _tc_final(p3, h3p, degp, b3p.reshape(1, -1))
